# baseline (speedup 1.0000x reference)
import functools

import jax
import jax.numpy as jnp
from jax import lax
from jax.experimental import pallas as pl
from jax.experimental.pallas import tpu as pltpu
from jax.experimental.pallas import tpu_sc as plsc

_B, _S, _D = 4, 2048, 1024
_H = 2048
_E = 8
_N = _B * _S
_NP = _N * 2
_T = 256
_NT = _NP // _T + _E
_NS = _NT * _T
_GB = 1024

_NW = 32
_CHUNK = _NP // _NW
_CV = _CHUNK // 16
_NTP = 80
_RC = 32
_TK = _N // _NW
_CB = 16


def _gating_body(x_ref, wg_ref, bg_ref, i0_ref, i1_ref, g0_ref, g1_ref,
                 psum_ref, csum_ref, kcnt_ref):
    i = pl.program_id(0)
    xb = x_ref[...]
    logits = jnp.dot(xb, wg_ref[...], preferred_element_type=jnp.float32)
    logits = logits + bg_ref[...]
    e_iota = lax.broadcasted_iota(jnp.int32, (_GB, _E), 1)
    l0 = jnp.max(logits, axis=1, keepdims=True)
    i0 = jnp.min(jnp.where(logits == l0, e_iota, _E), axis=1, keepdims=True)
    masked = jnp.where(e_iota == i0, -1e30, logits)
    l1 = jnp.max(masked, axis=1, keepdims=True)
    i1 = jnp.min(jnp.where(masked == l1, e_iota, _E), axis=1, keepdims=True)
    t = jnp.exp(l1 - l0)
    g0 = 1.0 / (1.0 + t)
    g1 = t / (1.0 + t)
    i0_ref[...] = i0.reshape(1, 1, _GB)
    i1_ref[...] = i1.reshape(1, 1, _GB)
    g0_ref[...] = g0.reshape(1, 1, _GB)
    g1_ref[...] = g1.reshape(1, 1, _GB)
    p = jnp.exp(logits - l0)
    p = p / jnp.sum(p, axis=1, keepdims=True)
    psum = jnp.sum(p, axis=0, keepdims=True)
    oh0 = (e_iota == i0).astype(jnp.float32)
    oh1 = (e_iota == i1).astype(jnp.float32)
    csum = jnp.sum(g0 * oh0 + g1 * oh1, axis=0, keepdims=True)
    ksum = jnp.sum((oh0 + oh1).astype(jnp.int32), axis=0, keepdims=True)

    @pl.when(i == 0)
    def _():
        psum_ref[...] = jnp.zeros_like(psum_ref)
        csum_ref[...] = jnp.zeros_like(csum_ref)
        kcnt_ref[...] = jnp.zeros_like(kcnt_ref)

    psum_ref[...] += psum
    csum_ref[...] += csum
    kcnt_ref[...] += ksum


def _gating(x2d, Wg, bg):
    nb = _N // _GB
    out = pl.pallas_call(
        _gating_body,
        grid=(nb,),
        in_specs=[
            pl.BlockSpec((_GB, _D), lambda i: (i, 0)),
            pl.BlockSpec((_D, _E), lambda i: (0, 0)),
            pl.BlockSpec((1, _E), lambda i: (0, 0)),
        ],
        out_specs=[
            pl.BlockSpec((1, 1, _GB), lambda i: (i, 0, 0)),
            pl.BlockSpec((1, 1, _GB), lambda i: (i, 0, 0)),
            pl.BlockSpec((1, 1, _GB), lambda i: (i, 0, 0)),
            pl.BlockSpec((1, 1, _GB), lambda i: (i, 0, 0)),
            pl.BlockSpec((1, _E), lambda i: (0, 0)),
            pl.BlockSpec((1, _E), lambda i: (0, 0)),
            pl.BlockSpec((1, _E), lambda i: (0, 0)),
        ],
        out_shape=[
            jax.ShapeDtypeStruct((nb, 1, _GB), jnp.int32),
            jax.ShapeDtypeStruct((nb, 1, _GB), jnp.int32),
            jax.ShapeDtypeStruct((nb, 1, _GB), jnp.float32),
            jax.ShapeDtypeStruct((nb, 1, _GB), jnp.float32),
            jax.ShapeDtypeStruct((1, _E), jnp.float32),
            jax.ShapeDtypeStruct((1, _E), jnp.float32),
            jax.ShapeDtypeStruct((1, _E), jnp.int32),
        ],
        compiler_params=pltpu.CompilerParams(
            dimension_semantics=("arbitrary",)),
    )(x2d, Wg, bg.reshape(1, _E))
    i0, i1, g0, g1, psum, csum, kcnt = out
    return (i0.reshape(_N), i1.reshape(_N), g0.reshape(_N), g1.reshape(_N),
            psum.reshape(_E), csum.reshape(_E), kcnt.reshape(_E))


def _take16(vec, idx):
    return lax.gather(
        vec, idx[:, None],
        lax.GatherDimensionNumbers(offset_dims=(), collapsed_slice_dims=(0,),
                                   start_index_map=(0,)),
        slice_sizes=(1,),
        mode=lax.GatherScatterMode.PROMISE_IN_BOUNDS)


def _route_body(ids_hbm, cnt_hbm, x_hbm, xs_hbm, pos_hbm, te_hbm,
                ids_v, cnt_v, pos_v, posidx_v, posidx2_v, xrow_v, xrow2_v,
                te_v, isem_a, isem_b, osem_a, osem_b):
    wid = lax.axis_index("s") * 2 + lax.axis_index("c")
    pltpu.sync_copy(ids_hbm, ids_v)
    pltpu.sync_copy(cnt_hbm, cnt_v)
    zeros16 = jnp.zeros((16,), jnp.int32)
    lanes = lax.iota(jnp.int32, 16)

    cnt = cnt_v[...]
    pc = ((cnt + (_T - 1)) >> 8) << 8
    bases = plsc.cumsum(pc) - pc
    ends = bases + pc

    base_j = wid * _CV

    def count_step(j, pre):
        ev = ids_v[pl.ds(j * 16, 16)]
        for e in range(_E):
            c = jnp.sum((ev == e).astype(jnp.int32))
            pre = pre + jnp.where(lanes == e, c, 0)
        return pre

    pre = lax.fori_loop(0, base_j, count_step, zeros16)

    tok0 = (wid * _CHUNK) % _N

    def pos_step(v, start):
        ev = ids_v[pl.ds((base_j + v) * 16, 16)]
        sv = _take16(start, ev)
        r = zeros16
        delta = zeros16
        for e in range(_E):
            m = ev == e
            mi = m.astype(jnp.int32)
            cs = plsc.cumsum(mi)
            r = jnp.where(m, cs, r)
            delta = delta + jnp.where(lanes == e, jnp.sum(mi), 0)
        pvec = sv + r - 1
        pos_v[pl.ds(v * 16, 16)] = pvec
        return start + delta

    lax.fori_loop(0, _CV, pos_step, bases + pre)
    pltpu.sync_copy(pos_v, pos_hbm.at[pl.ds(wid * _CHUNK, _CHUNK)])

    nch = _CHUNK // _RC
    xbufs = (xrow_v, xrow2_v)
    pbufs = (posidx_v, posidx2_v)
    isems = (isem_a, isem_b)
    osems = (osem_a, osem_b)

    def stage(g, b):
        pltpu.async_copy(
            x_hbm.at[pl.ds(tok0 + g * _RC, _RC)], xbufs[b], isems[b])

    stage(0, 0)
    stage(1, 1)
    for g in range(nch):
        b = g % 2
        pltpu.make_async_copy(
            x_hbm.at[pl.ds(tok0, _RC)], xbufs[b], isems[b]).wait()
        for q in range(_RC // 16):
            pbufs[b][pl.ds(q * 16, 16)] = pos_v[pl.ds(g * _RC + q * 16, 16)]
        pltpu.async_copy(xbufs[b], xs_hbm.at[pbufs[b]], osems[b])
        pltpu.make_async_copy(
            x_hbm.at[pl.ds(tok0, _RC)], xbufs[b], osems[b]).wait()
        if g + 2 < nch:
            stage(g + 2, b)

    @pl.when(wid == 0)
    def _():
        for j in range(_NTP // 16):
            tv = (lanes + j * 16) * _T
            acc = zeros16
            for e in range(_E):
                end_e = _take16(ends, jnp.full((16,), e, jnp.int32))
                acc += (tv >= end_e).astype(jnp.int32)
            te_v[pl.ds(j * 16, 16)] = acc
        pltpu.sync_copy(te_v, te_hbm)


def _route(ids, counts16, x2d):
    f = pl.kernel(
        _route_body,
        mesh=plsc.VectorSubcoreMesh(core_axis_name="c", subcore_axis_name="s"),
        out_type=[
            jax.ShapeDtypeStruct((_NS, _D), jnp.float32),
            jax.ShapeDtypeStruct((_NP,), jnp.int32),
            jax.ShapeDtypeStruct((_NTP,), jnp.int32),
        ],
        scratch_types=[
            pltpu.VMEM((_NP,), jnp.int32),
            pltpu.VMEM((16,), jnp.int32),
            pltpu.VMEM((_CHUNK,), jnp.int32),
            pltpu.VMEM((_RC,), jnp.int32),
            pltpu.VMEM((_RC,), jnp.int32),
            pltpu.VMEM((_RC, _D), jnp.float32),
            pltpu.VMEM((_RC, _D), jnp.float32),
            pltpu.VMEM((_NTP,), jnp.int32),
            pltpu.SemaphoreType.DMA,
            pltpu.SemaphoreType.DMA,
            pltpu.SemaphoreType.DMA,
            pltpu.SemaphoreType.DMA,
        ],
        compiler_params=pltpu.CompilerParams(needs_layout_passes=False),
    )
    return f(ids, counts16, x2d)


def _clampe(i):
    return jnp.minimum(i, _E - 1)


def _mm12_body(te_ref, x_ref, w1_ref, b1_ref, w2_ref, b2_ref, o_ref):
    t = pl.program_id(0)

    @pl.when(te_ref[t] < _E)
    def _():
        h1 = jnp.dot(x_ref[...], w1_ref[0],
                     preferred_element_type=jnp.float32)
        h1 = jnp.maximum(h1 + b1_ref[0], 0.0)
        h2 = jnp.dot(h1, w2_ref[0], preferred_element_type=jnp.float32)
        o_ref[...] = jnp.maximum(h2 + b2_ref[0], 0.0)


def _grouped_mm12(te, xs, W1, b1, W2, b2):
    return pl.pallas_call(
        _mm12_body,
        grid_spec=pltpu.PrefetchScalarGridSpec(
            num_scalar_prefetch=1,
            grid=(_NT,),
            in_specs=[
                pl.BlockSpec((_T, _D), lambda t, s: (t, 0)),
                pl.BlockSpec((1, _D, _H), lambda t, s: (_clampe(s[t]), 0, 0)),
                pl.BlockSpec((1, 1, _H), lambda t, s: (_clampe(s[t]), 0, 0)),
                pl.BlockSpec((1, _H, _H), lambda t, s: (_clampe(s[t]), 0, 0)),
                pl.BlockSpec((1, 1, _H), lambda t, s: (_clampe(s[t]), 0, 0)),
            ],
            out_specs=pl.BlockSpec((_T, _H), lambda t, s: (t, 0)),
        ),
        out_shape=jax.ShapeDtypeStruct((_NS, _H), jnp.float32),
        compiler_params=pltpu.CompilerParams(
            dimension_semantics=("arbitrary",)),
    )(te, xs, W1, b1.reshape(_E, 1, _H), W2, b2.reshape(_E, 1, _H))


def _mm3_body(te_ref, x_ref, w_ref, b_ref, o_ref):
    t = pl.program_id(0)

    @pl.when(te_ref[t] < _E)
    def _():
        acc = jnp.dot(x_ref[...], w_ref[0],
                      preferred_element_type=jnp.float32)
        o_ref[...] = acc + b_ref[0]


def _grouped_mm3(te, h, W3, b3):
    return pl.pallas_call(
        _mm3_body,
        grid_spec=pltpu.PrefetchScalarGridSpec(
            num_scalar_prefetch=1,
            grid=(_NT,),
            in_specs=[
                pl.BlockSpec((_T, _H), lambda t, s: (t, 0)),
                pl.BlockSpec((1, _H, _D), lambda t, s: (_clampe(s[t]), 0, 0)),
                pl.BlockSpec((1, 1, _D), lambda t, s: (_clampe(s[t]), 0, 0)),
            ],
            out_specs=pl.BlockSpec((_T, _D), lambda t, s: (t, 0)),
        ),
        out_shape=jax.ShapeDtypeStruct((_NS, _D), jnp.float32),
        compiler_params=pltpu.CompilerParams(
            dimension_semantics=("arbitrary",)),
    )(te, h, W3, b3.reshape(_E, 1, _D))


def _combine_body(y_hbm, pos_hbm, g0_hbm, g1_hbm, out_hbm,
                  p0_v, p1_v, g0_v, g1_v, a_v, a2_v, b_v, b2_v, o_v, o2_v,
                  ia_a, ia_b, ib_a, ib_b, os_a, os_b):
    wid = lax.axis_index("s") * 2 + lax.axis_index("c")
    t0 = wid * _TK
    pltpu.sync_copy(pos_hbm.at[pl.ds(t0, _TK)], p0_v)
    pltpu.sync_copy(pos_hbm.at[pl.ds(_N + t0, _TK)], p1_v)
    pltpu.sync_copy(g0_hbm.at[pl.ds(t0, _TK)], g0_v)
    pltpu.sync_copy(g1_hbm.at[pl.ds(t0, _TK)], g1_v)

    nch = _TK // _CB
    abufs, bbufs, obufs = (a_v, a2_v), (b_v, b2_v), (o_v, o2_v)
    ias, ibs, oss = (ia_a, ia_b), (ib_a, ib_b), (os_a, os_b)

    def gathers(g, b):
        pltpu.async_copy(
            y_hbm.at[p0_v.at[pl.ds(g * _CB, _CB)]], abufs[b], ias[b])
        pltpu.async_copy(
            y_hbm.at[p1_v.at[pl.ds(g * _CB, _CB)]], bbufs[b], ibs[b])

    gathers(0, 0)
    gathers(1, 1)
    for g in range(nch):
        b = g % 2
        if g >= 2:
            pltpu.make_async_copy(
                obufs[b], out_hbm.at[pl.ds(t0, _CB)], oss[b]).wait()
        pltpu.make_async_copy(
            y_hbm.at[p0_v.at[pl.ds(0, _CB)]], abufs[b], ias[b]).wait()
        pltpu.make_async_copy(
            y_hbm.at[p1_v.at[pl.ds(0, _CB)]], bbufs[b], ibs[b]).wait()
        gv0 = g0_v[pl.ds(g * _CB, 16)]
        gv1 = g1_v[pl.ds(g * _CB, 16)]

        def tok_step(j, carry, b=b, gv0=gv0, gv1=gv1):
            lane = jnp.full((16,), j, jnp.int32)
            g0s = _take16(gv0, lane)
            g1s = _take16(gv1, lane)

            def seg_step(sq, carry2):
                for u in range(8):
                    sl = pl.ds(sq * 128 + u * 16, 16)
                    av = abufs[b][j, sl]
                    bv = bbufs[b][j, sl]
                    obufs[b][j, sl] = g0s * av + g1s * bv
                return carry2

            lax.fori_loop(0, _D // 128, seg_step, 0)
            return carry

        lax.fori_loop(0, _CB, tok_step, 0)
        if g + 2 < nch:
            gathers(g + 2, b)
        pltpu.async_copy(
            obufs[b], out_hbm.at[pl.ds(t0 + g * _CB, _CB)], oss[b])

    for b in range(2):
        pltpu.make_async_copy(
            obufs[b], out_hbm.at[pl.ds(t0, _CB)], oss[b]).wait()


def _combine(y, pos, g0, g1):
    f = pl.kernel(
        _combine_body,
        mesh=plsc.VectorSubcoreMesh(core_axis_name="c", subcore_axis_name="s"),
        out_type=jax.ShapeDtypeStruct((_N, _D), jnp.float32),
        scratch_types=[
            pltpu.VMEM((_TK,), jnp.int32),
            pltpu.VMEM((_TK,), jnp.int32),
            pltpu.VMEM((_TK,), jnp.float32),
            pltpu.VMEM((_TK,), jnp.float32),
            pltpu.VMEM((_CB, _D), jnp.float32),
            pltpu.VMEM((_CB, _D), jnp.float32),
            pltpu.VMEM((_CB, _D), jnp.float32),
            pltpu.VMEM((_CB, _D), jnp.float32),
            pltpu.VMEM((_CB, _D), jnp.float32),
            pltpu.VMEM((_CB, _D), jnp.float32),
            pltpu.SemaphoreType.DMA,
            pltpu.SemaphoreType.DMA,
            pltpu.SemaphoreType.DMA,
            pltpu.SemaphoreType.DMA,
            pltpu.SemaphoreType.DMA,
            pltpu.SemaphoreType.DMA,
        ],
        compiler_params=pltpu.CompilerParams(needs_layout_passes=False),
    )
    return f(y, pos, g0, g1)


def kernel(x, W1, b1, W2, b2, W3, b3, Wg, bg):
    x2d = x.reshape(_N, _D)
    i0, i1, g0, g1, psum, csum, kcnt = _gating(x2d, Wg, bg)

    ids = jnp.concatenate([i0, i1])
    counts16 = jnp.concatenate([kcnt, jnp.zeros((16 - _E,), jnp.int32)])
    xs, pos, te_pad = _route(ids, counts16, x2d)
    te = te_pad[:_NT]

    h = _grouped_mm12(te, xs, W1, b1, W2, b2)
    y = _grouped_mm3(te, h, W3, b3)

    out2d = _combine(y, pos, g0, g1)

    avg_probs = psum / _N
    avg_counts = csum / _N
    lb = 0.01 * _E * jnp.sum(avg_probs * avg_counts)
    ent = -jnp.sum(avg_probs * jnp.log(avg_probs + 1e-08))
    return (out2d.reshape(_B, _S, _D), lb, avg_counts, ent)


# speedup vs baseline: 1.1203x; 1.1203x over previous
import functools

import jax
import jax.numpy as jnp
from jax import lax
from jax.experimental import pallas as pl
from jax.experimental.pallas import tpu as pltpu
from jax.experimental.pallas import tpu_sc as plsc

_B, _S, _D = 4, 2048, 1024
_H = 2048
_E = 8
_N = _B * _S
_NP = _N * 2
_T = 256
_NT = _NP // _T + _E
_NS = _NT * _T
_GB = 1024

_NW = 32
_CHUNK = _NP // _NW
_CV = _CHUNK // 16
_NTP = 80
_RC = 32
_TK = _N // _NW
_CB = 16


def _gating_body(x_ref, wg_ref, bg_ref, i0_ref, i1_ref, g0_ref, g1_ref,
                 psum_ref, csum_ref, kcnt_ref):
    i = pl.program_id(0)
    xb = x_ref[...]
    logits = jnp.dot(xb, wg_ref[...], preferred_element_type=jnp.float32)
    logits = logits + bg_ref[...]
    e_iota = lax.broadcasted_iota(jnp.int32, (_GB, _E), 1)
    l0 = jnp.max(logits, axis=1, keepdims=True)
    i0 = jnp.min(jnp.where(logits == l0, e_iota, _E), axis=1, keepdims=True)
    masked = jnp.where(e_iota == i0, -1e30, logits)
    l1 = jnp.max(masked, axis=1, keepdims=True)
    i1 = jnp.min(jnp.where(masked == l1, e_iota, _E), axis=1, keepdims=True)
    t = jnp.exp(l1 - l0)
    g0 = 1.0 / (1.0 + t)
    g1 = t / (1.0 + t)
    i0_ref[...] = i0.reshape(1, 1, _GB)
    i1_ref[...] = i1.reshape(1, 1, _GB)
    g0_ref[...] = g0.reshape(1, 1, _GB)
    g1_ref[...] = g1.reshape(1, 1, _GB)
    p = jnp.exp(logits - l0)
    p = p / jnp.sum(p, axis=1, keepdims=True)
    psum = jnp.sum(p, axis=0, keepdims=True)
    oh0 = (e_iota == i0).astype(jnp.float32)
    oh1 = (e_iota == i1).astype(jnp.float32)
    csum = jnp.sum(g0 * oh0 + g1 * oh1, axis=0, keepdims=True)
    ksum = jnp.sum((oh0 + oh1).astype(jnp.int32), axis=0, keepdims=True)

    @pl.when(i == 0)
    def _():
        psum_ref[...] = jnp.zeros_like(psum_ref)
        csum_ref[...] = jnp.zeros_like(csum_ref)
        kcnt_ref[...] = jnp.zeros_like(kcnt_ref)

    psum_ref[...] += psum
    csum_ref[...] += csum
    kcnt_ref[...] += ksum


def _gating(x2d, Wg, bg):
    nb = _N // _GB
    out = pl.pallas_call(
        _gating_body,
        grid=(nb,),
        in_specs=[
            pl.BlockSpec((_GB, _D), lambda i: (i, 0)),
            pl.BlockSpec((_D, _E), lambda i: (0, 0)),
            pl.BlockSpec((1, _E), lambda i: (0, 0)),
        ],
        out_specs=[
            pl.BlockSpec((1, 1, _GB), lambda i: (i, 0, 0)),
            pl.BlockSpec((1, 1, _GB), lambda i: (i, 0, 0)),
            pl.BlockSpec((1, 1, _GB), lambda i: (i, 0, 0)),
            pl.BlockSpec((1, 1, _GB), lambda i: (i, 0, 0)),
            pl.BlockSpec((1, _E), lambda i: (0, 0)),
            pl.BlockSpec((1, _E), lambda i: (0, 0)),
            pl.BlockSpec((1, _E), lambda i: (0, 0)),
        ],
        out_shape=[
            jax.ShapeDtypeStruct((nb, 1, _GB), jnp.int32),
            jax.ShapeDtypeStruct((nb, 1, _GB), jnp.int32),
            jax.ShapeDtypeStruct((nb, 1, _GB), jnp.float32),
            jax.ShapeDtypeStruct((nb, 1, _GB), jnp.float32),
            jax.ShapeDtypeStruct((1, _E), jnp.float32),
            jax.ShapeDtypeStruct((1, _E), jnp.float32),
            jax.ShapeDtypeStruct((1, _E), jnp.int32),
        ],
        compiler_params=pltpu.CompilerParams(
            dimension_semantics=("arbitrary",)),
    )(x2d, Wg, bg.reshape(1, _E))
    i0, i1, g0, g1, psum, csum, kcnt = out
    return (i0.reshape(_N), i1.reshape(_N), g0.reshape(_N), g1.reshape(_N),
            psum.reshape(_E), csum.reshape(_E), kcnt.reshape(_E))


def _take16(vec, idx):
    return lax.gather(
        vec, idx[:, None],
        lax.GatherDimensionNumbers(offset_dims=(), collapsed_slice_dims=(0,),
                                   start_index_map=(0,)),
        slice_sizes=(1,),
        mode=lax.GatherScatterMode.PROMISE_IN_BOUNDS)


def _route_body(ids_hbm, cnt_hbm, x_hbm, xs_hbm, pos_hbm, te_hbm,
                ids_v, cnt_v, pos_v, posidx_v, posidx2_v, xrow_v, xrow2_v,
                te_v, isem_a, isem_b, osem_a, osem_b):
    wid = lax.axis_index("s") * 2 + lax.axis_index("c")
    pltpu.sync_copy(ids_hbm, ids_v)
    pltpu.sync_copy(cnt_hbm, cnt_v)
    zeros16 = jnp.zeros((16,), jnp.int32)
    lanes = lax.iota(jnp.int32, 16)

    cnt = cnt_v[...]
    pc = ((cnt + (_T - 1)) >> 8) << 8
    bases = plsc.cumsum(pc) - pc
    ends = bases + pc

    base_j = wid * _CV

    def count_step(j, pre):
        ev = ids_v[pl.ds(j * 16, 16)]
        for e in range(_E):
            c = jnp.sum((ev == e).astype(jnp.int32))
            pre = pre + jnp.where(lanes == e, c, 0)
        return pre

    pre = lax.fori_loop(0, base_j, count_step, zeros16)

    tok0 = (wid * _CHUNK) % _N

    def pos_step(v, start):
        ev = ids_v[pl.ds((base_j + v) * 16, 16)]
        sv = _take16(start, ev)
        r = zeros16
        delta = zeros16
        for e in range(_E):
            m = ev == e
            mi = m.astype(jnp.int32)
            cs = plsc.cumsum(mi)
            r = jnp.where(m, cs, r)
            delta = delta + jnp.where(lanes == e, jnp.sum(mi), 0)
        pvec = sv + r - 1
        pos_v[pl.ds(v * 16, 16)] = pvec
        return start + delta

    lax.fori_loop(0, _CV, pos_step, bases + pre)
    pltpu.sync_copy(pos_v, pos_hbm.at[pl.ds(wid * _CHUNK, _CHUNK)])

    nch = _CHUNK // _RC
    xbufs = (xrow_v, xrow2_v)
    pbufs = (posidx_v, posidx2_v)
    isems = (isem_a, isem_b)
    osems = (osem_a, osem_b)

    def stage(g, b):
        pltpu.async_copy(
            x_hbm.at[pl.ds(tok0 + g * _RC, _RC)], xbufs[b], isems[b])

    stage(0, 0)
    stage(1, 1)
    for g in range(nch):
        b = g % 2
        pltpu.make_async_copy(
            x_hbm.at[pl.ds(tok0, _RC)], xbufs[b], isems[b]).wait()
        for q in range(_RC // 16):
            pbufs[b][pl.ds(q * 16, 16)] = pos_v[pl.ds(g * _RC + q * 16, 16)]
        pltpu.async_copy(xbufs[b], xs_hbm.at[pbufs[b]], osems[b])
        pltpu.make_async_copy(
            x_hbm.at[pl.ds(tok0, _RC)], xbufs[b], osems[b]).wait()
        if g + 2 < nch:
            stage(g + 2, b)

    @pl.when(wid == 0)
    def _():
        for j in range(_NTP // 16):
            tv = (lanes + j * 16) * _T
            acc = zeros16
            for e in range(_E):
                end_e = _take16(ends, jnp.full((16,), e, jnp.int32))
                acc += (tv >= end_e).astype(jnp.int32)
            te_v[pl.ds(j * 16, 16)] = acc
        pltpu.sync_copy(te_v, te_hbm)


def _route(ids, counts16, x2d):
    f = pl.kernel(
        _route_body,
        mesh=plsc.VectorSubcoreMesh(core_axis_name="c", subcore_axis_name="s"),
        out_type=[
            jax.ShapeDtypeStruct((_NS, _D), jnp.float32),
            jax.ShapeDtypeStruct((_NP,), jnp.int32),
            jax.ShapeDtypeStruct((_NTP,), jnp.int32),
        ],
        scratch_types=[
            pltpu.VMEM((_NP,), jnp.int32),
            pltpu.VMEM((16,), jnp.int32),
            pltpu.VMEM((_CHUNK,), jnp.int32),
            pltpu.VMEM((_RC,), jnp.int32),
            pltpu.VMEM((_RC,), jnp.int32),
            pltpu.VMEM((_RC, _D), jnp.float32),
            pltpu.VMEM((_RC, _D), jnp.float32),
            pltpu.VMEM((_NTP,), jnp.int32),
            pltpu.SemaphoreType.DMA,
            pltpu.SemaphoreType.DMA,
            pltpu.SemaphoreType.DMA,
            pltpu.SemaphoreType.DMA,
        ],
        compiler_params=pltpu.CompilerParams(needs_layout_passes=False),
    )
    return f(ids, counts16, x2d)


def _clampe(i):
    return jnp.minimum(i, _E - 1)


def _mm12_body(te_ref, x_ref, w1_ref, b1_ref, w2_ref, b2_ref, o_ref):
    t = pl.program_id(0)

    @pl.when(te_ref[t] < _E)
    def _():
        h1 = jnp.dot(x_ref[...], w1_ref[0],
                     preferred_element_type=jnp.float32)
        h1 = jnp.maximum(h1 + b1_ref[0], 0.0)
        h2 = jnp.dot(h1, w2_ref[0], preferred_element_type=jnp.float32)
        o_ref[...] = jnp.maximum(h2 + b2_ref[0], 0.0).astype(jnp.bfloat16)


def _grouped_mm12(te, xs, W1, b1, W2, b2):
    return pl.pallas_call(
        _mm12_body,
        grid_spec=pltpu.PrefetchScalarGridSpec(
            num_scalar_prefetch=1,
            grid=(_NT,),
            in_specs=[
                pl.BlockSpec((_T, _D), lambda t, s: (t, 0)),
                pl.BlockSpec((1, _D, _H), lambda t, s: (_clampe(s[t]), 0, 0)),
                pl.BlockSpec((1, 1, _H), lambda t, s: (_clampe(s[t]), 0, 0)),
                pl.BlockSpec((1, _H, _H), lambda t, s: (_clampe(s[t]), 0, 0)),
                pl.BlockSpec((1, 1, _H), lambda t, s: (_clampe(s[t]), 0, 0)),
            ],
            out_specs=pl.BlockSpec((_T, _H), lambda t, s: (t, 0)),
        ),
        out_shape=jax.ShapeDtypeStruct((_NS, _H), jnp.bfloat16),
        compiler_params=pltpu.CompilerParams(
            dimension_semantics=("arbitrary",)),
    )(te, xs, W1, b1.reshape(_E, 1, _H), W2, b2.reshape(_E, 1, _H))


def _mm3_body(te_ref, x_ref, w_ref, b_ref, o_ref):
    t = pl.program_id(0)

    @pl.when(te_ref[t] < _E)
    def _():
        acc = jnp.dot(x_ref[...].astype(jnp.float32), w_ref[0],
                      preferred_element_type=jnp.float32)
        o_ref[...] = acc + b_ref[0]


def _grouped_mm3(te, h, W3, b3):
    return pl.pallas_call(
        _mm3_body,
        grid_spec=pltpu.PrefetchScalarGridSpec(
            num_scalar_prefetch=1,
            grid=(_NT,),
            in_specs=[
                pl.BlockSpec((_T, _H), lambda t, s: (t, 0)),
                pl.BlockSpec((1, _H, _D), lambda t, s: (_clampe(s[t]), 0, 0)),
                pl.BlockSpec((1, 1, _D), lambda t, s: (_clampe(s[t]), 0, 0)),
            ],
            out_specs=pl.BlockSpec((_T, _D), lambda t, s: (t, 0)),
        ),
        out_shape=jax.ShapeDtypeStruct((_NS, _D), jnp.float32),
        compiler_params=pltpu.CompilerParams(
            dimension_semantics=("arbitrary",)),
    )(te, h, W3, b3.reshape(_E, 1, _D))


def _combine_body(y_hbm, pos_hbm, g0_hbm, g1_hbm, out_hbm,
                  p0_v, p1_v, pcat_v, g0_v, g1_v, ab_v, ab2_v, o_v, o2_v,
                  iab_a, iab_b, os_a, os_b):
    wid = lax.axis_index("s") * 2 + lax.axis_index("c")
    t0 = wid * _TK
    pltpu.sync_copy(pos_hbm.at[pl.ds(t0, _TK)], p0_v)
    pltpu.sync_copy(pos_hbm.at[pl.ds(_N + t0, _TK)], p1_v)
    pltpu.sync_copy(g0_hbm.at[pl.ds(t0, _TK)], g0_v)
    pltpu.sync_copy(g1_hbm.at[pl.ds(t0, _TK)], g1_v)

    nch = _TK // _CB
    for g in range(nch):
        pcat_v[pl.ds(g * 32, 16)] = p0_v[pl.ds(g * _CB, 16)]
        pcat_v[pl.ds(g * 32 + 16, 16)] = p1_v[pl.ds(g * _CB, 16)]

    abufs, obufs = (ab_v, ab2_v), (o_v, o2_v)
    iabs, oss = (iab_a, iab_b), (os_a, os_b)

    def gather(g, b):
        pltpu.async_copy(
            y_hbm.at[pcat_v.at[pl.ds(g * 32, 32)]], abufs[b], iabs[b])

    gather(0, 0)
    gather(1, 1)
    for g in range(nch):
        b = g % 2
        if g >= 2:
            pltpu.make_async_copy(
                obufs[b], out_hbm.at[pl.ds(t0, _CB)], oss[b]).wait()
        pltpu.make_async_copy(
            y_hbm.at[pcat_v.at[pl.ds(0, 32)]], abufs[b], iabs[b]).wait()

        def tok_step(j, carry, b=b, g=g):
            lane = jnp.full((16,), j, jnp.int32)
            g0s = _take16(g0_v[pl.ds(g * _CB, 16)], lane)
            g1s = _take16(g1_v[pl.ds(g * _CB, 16)], lane)
            for seg in range(_D // 16):
                sl = pl.ds(seg * 16, 16)
                av = abufs[b][j, sl]
                bv = abufs[b][j + 16, sl]
                obufs[b][j, sl] = g0s * av + g1s * bv
            return carry

        lax.fori_loop(0, _CB, tok_step, 0)
        if g + 2 < nch:
            gather(g + 2, b)
        pltpu.async_copy(
            obufs[b], out_hbm.at[pl.ds(t0 + g * _CB, _CB)], oss[b])

    for b in range(2):
        pltpu.make_async_copy(
            obufs[b], out_hbm.at[pl.ds(t0, _CB)], oss[b]).wait()


def _combine(y, pos, g0, g1):
    f = pl.kernel(
        _combine_body,
        mesh=plsc.VectorSubcoreMesh(core_axis_name="c", subcore_axis_name="s"),
        out_type=jax.ShapeDtypeStruct((_N, _D), jnp.float32),
        scratch_types=[
            pltpu.VMEM((_TK,), jnp.int32),
            pltpu.VMEM((_TK,), jnp.int32),
            pltpu.VMEM((2 * _TK,), jnp.int32),
            pltpu.VMEM((_TK,), jnp.float32),
            pltpu.VMEM((_TK,), jnp.float32),
            pltpu.VMEM((2 * _CB, _D), jnp.float32),
            pltpu.VMEM((2 * _CB, _D), jnp.float32),
            pltpu.VMEM((_CB, _D), jnp.float32),
            pltpu.VMEM((_CB, _D), jnp.float32),
            pltpu.SemaphoreType.DMA,
            pltpu.SemaphoreType.DMA,
            pltpu.SemaphoreType.DMA,
            pltpu.SemaphoreType.DMA,
        ],
        compiler_params=pltpu.CompilerParams(needs_layout_passes=False),
    )
    return f(y, pos, g0, g1)


def kernel(x, W1, b1, W2, b2, W3, b3, Wg, bg):
    x2d = x.reshape(_N, _D)
    i0, i1, g0, g1, psum, csum, kcnt = _gating(x2d, Wg, bg)

    ids = jnp.concatenate([i0, i1])
    counts16 = jnp.concatenate([kcnt, jnp.zeros((16 - _E,), jnp.int32)])
    xs, pos, te_pad = _route(ids, counts16, x2d)
    te = te_pad[:_NT]

    h = _grouped_mm12(te, xs, W1, b1, W2, b2)
    y = _grouped_mm3(te, h, W3, b3)

    out2d = _combine(y, pos, g0, g1)

    avg_probs = psum / _N
    avg_counts = csum / _N
    lb = 0.01 * _E * jnp.sum(avg_probs * avg_counts)
    ent = -jnp.sum(avg_probs * jnp.log(avg_probs + 1e-08))
    return (out2d.reshape(_B, _S, _D), lb, avg_counts, ent)
